# baseline (speedup 1.0000x reference)
import functools

import jax
import jax.numpy as jnp
from jax import lax
from jax.experimental import pallas as pl
from jax.experimental.pallas import tpu as pltpu
from jax.experimental.pallas import tpu_sc as plsc

_L1, _L2, _ORBIT, _PDIM, _EDIM = 16, 16, 2, 4, 64
_J = _L1 * _L2 * _ORBIT
_PAIRS = _J // 2
_COMBO = _PDIM * _PDIM
_W = 2 * _EDIM
_LANES = 16
_CHUNK = _PAIRS
_NSUB = _CHUNK // 128


def kernel(x, parameter):
    b = x.shape[0]
    rows = b * _PAIRS
    xe = x[..., 0].reshape(rows)
    xo = x[..., 1].reshape(rows)

    p4 = parameter.reshape(_PAIRS, _ORBIT, _PDIM, _EDIM)
    ctab = jnp.concatenate(
        [
            jnp.broadcast_to(p4[:, 0, :, None, :], (_PAIRS, _PDIM, _PDIM, _EDIM)),
            jnp.broadcast_to(p4[:, 1, None, :, :], (_PAIRS, _PDIM, _PDIM, _EDIM)),
        ],
        axis=-1,
    ).reshape(_PAIRS * _COMBO, _W)

    info = plsc.get_sparse_core_info()
    num_workers = info.num_cores * info.num_subcores
    rows_per_w = rows // num_workers
    n_chunks = rows_per_w // _CHUNK

    mesh = plsc.VectorSubcoreMesh(core_axis_name="c", subcore_axis_name="s")

    @functools.partial(
        pl.kernel,
        mesh=mesh,
        out_type=jax.ShapeDtypeStruct((rows, _W), jnp.float32),
        scratch_types=[
            pltpu.VMEM((2, _CHUNK), jnp.int32),
            pltpu.VMEM((2, _CHUNK), jnp.int32),
            pltpu.VMEM((2 * _NSUB, 128), jnp.int32),
            pltpu.VMEM((2, _CHUNK, _W), jnp.float32),
            pltpu.VMEM_SHARED((_PAIRS * _COMBO, _W), jnp.float32),
            pltpu.SemaphoreType.DMA,
            pltpu.SemaphoreType.DMA,
            pltpu.SemaphoreType.DMA,
        ],
    )
    def emb(xe_hbm, xo_hbm, tab_hbm, out_hbm,
            xe_v, xo_v, idx_v, rows_v, tab_sh, sem_x, sem_g, sem_w):
        sid = lax.axis_index("s")
        wid = sid * info.num_cores + lax.axis_index("c")
        base = wid * rows_per_w

        @pl.when(sid == 0)
        def _stage_table():
            pltpu.sync_copy(tab_hbm, tab_sh)

        plsc.subcore_barrier()

        def start_x(c, p):
            rb = base + c * _CHUNK
            pltpu.async_copy(xe_hbm.at[pl.ds(rb, _CHUNK)], xe_v.at[p], sem_x)
            pltpu.async_copy(xo_hbm.at[pl.ds(rb, _CHUNK)], xo_v.at[p], sem_x)

        def wait_x(p):
            pltpu.make_async_copy(xe_hbm.at[pl.ds(0, _CHUNK)], xe_v.at[p], sem_x).wait()
            pltpu.make_async_copy(xo_hbm.at[pl.ds(0, _CHUNK)], xo_v.at[p], sem_x).wait()

        def wait_w(p):
            pltpu.make_async_copy(
                rows_v.at[p], out_hbm.at[pl.ds(0, _CHUNK)], sem_w
            ).wait()

        start_x(0, 0)

        def chunk_body(c, carry):
            p = lax.rem(c, 2)
            rb = base + c * _CHUNK

            @pl.when(c + 1 < n_chunks)
            def _prefetch():
                start_x(c + 1, 1 - p)

            wait_x(p)

            for k in range(_NSUB):
                idx_row = idx_v.at[p * _NSUB + k]
                for i in range(128 // _LANES):
                    off = k * 128 + i * _LANES
                    e = xe_v[p, pl.ds(off, _LANES)]
                    o = xo_v[p, pl.ds(off, _LANES)]
                    pair = lax.iota(jnp.int32, _LANES) + off
                    idx_row[pl.ds(i * _LANES, _LANES)] = pair * _COMBO + e * _PDIM + o

            @pl.when(c >= 2)
            def _drain_prev_write():
                wait_w(p)

            for k in range(_NSUB):
                pltpu.async_copy(
                    tab_sh.at[idx_v.at[p * _NSUB + k]],
                    rows_v.at[(p, pl.ds(k * 128, 128))],
                    sem_g,
                )
            for k in range(_NSUB):
                pltpu.make_async_copy(
                    tab_sh.at[idx_v.at[p * _NSUB + k]],
                    rows_v.at[(p, pl.ds(k * 128, 128))],
                    sem_g,
                ).wait()

            pltpu.async_copy(rows_v.at[p], out_hbm.at[pl.ds(rb, _CHUNK)], sem_w)
            return carry

        lax.fori_loop(0, n_chunks, chunk_body, 0)

        wait_w(lax.rem(n_chunks - 2, 2))
        wait_w(lax.rem(n_chunks - 1, 2))

    dense = emb(xe, xo, ctab)

    blk_b = 8
    def repack(i_ref, o_ref):
        xb = i_ref[...]
        a = xb[:, :_EDIM]
        bb = xb[:, _EDIM:]
        n2 = 2 * xb.shape[0]
        parity = lax.broadcasted_iota(jnp.int32, (n2, _EDIM), 0) % 2
        a2 = jnp.repeat(a, 2, axis=0)
        b2 = jnp.repeat(bb, 2, axis=0)
        o_ref[...] = jnp.where(parity == 0, a2, b2)

    out = pl.pallas_call(
        repack,
        grid=(b // blk_b,),
        in_specs=[pl.BlockSpec((blk_b * _PAIRS, _W), lambda i: (i, 0))],
        out_specs=pl.BlockSpec((blk_b * _J, _EDIM), lambda i: (i, 0)),
        out_shape=jax.ShapeDtypeStruct((b * _J, _EDIM), jnp.float32),
    )(dense)
    return out.reshape(b, _J, _EDIM)
